# baseline (speedup 1.0000x reference)
import functools

import jax
import jax.numpy as jnp
from jax import lax
from jax.experimental import pallas as pl
from jax.experimental.pallas import tpu as pltpu
from jax.experimental.pallas import tpu_sc as plsc

N = 10000
E = 320000
D = 128

NC = 2
NS = 16
NW = NC * NS
B = 128
CPT = 80
EPT = CPT * B
PADE = NW * EPT
PADC = PADE // B
S = 8
NPAD = 10240
SLC = NPAD // NS
NPADO = 10112
SLCO = NPADO // NS

_mesh = plsc.VectorSubcoreMesh(core_axis_name="c", subcore_axis_name="s")


_BLK = 1000
_NBLK = N // _BLK


def _prep_body(x_ref, emb_ref, w_ref, ai_ref, aj_ref, h_ref,
               sai_ref, saj_ref):
    h = jnp.dot(x_ref[...], w_ref[...], preferred_element_type=jnp.float32)
    h_ref[...] = h
    e = emb_ref[...]
    sai = jnp.sum(h * ai_ref[0, :][None, :], axis=1) + jnp.sum(
        e * ai_ref[1, :][None, :], axis=1)
    saj = jnp.sum(h * aj_ref[0, :][None, :], axis=1) + jnp.sum(
        e * aj_ref[1, :][None, :], axis=1)
    sai_ref[...] = sai.reshape(1, 1, _BLK)
    saj_ref[...] = saj.reshape(1, 1, _BLK)


def _prep(x, emb, w, ai2, aj2):
    return pl.pallas_call(
        _prep_body,
        grid=(_NBLK,),
        in_specs=[
            pl.BlockSpec((_BLK, D), lambda i: (i, 0)),
            pl.BlockSpec((_BLK, D), lambda i: (i, 0)),
            pl.BlockSpec((D, D), lambda i: (0, 0)),
            pl.BlockSpec((2, D), lambda i: (0, 0)),
            pl.BlockSpec((2, D), lambda i: (0, 0)),
        ],
        out_specs=[
            pl.BlockSpec((_BLK, D), lambda i: (i, 0)),
            pl.BlockSpec((1, 1, _BLK), lambda i: (i, 0, 0)),
            pl.BlockSpec((1, 1, _BLK), lambda i: (i, 0, 0)),
        ],
        out_shape=[
            jax.ShapeDtypeStruct((N, D), jnp.float32),
            jax.ShapeDtypeStruct((_NBLK, 1, _BLK), jnp.float32),
            jax.ShapeDtypeStruct((_NBLK, 1, _BLK), jnp.float32),
        ],
    )(x, emb, w, ai2, aj2)


@functools.partial(
    pl.kernel,
    out_type=[
        jax.ShapeDtypeStruct((PADC, B), jnp.float32),
        jax.ShapeDtypeStruct((NC * NPAD,), jnp.float32),
    ],
    mesh=_mesh,
    compiler_params=pltpu.CompilerParams(needs_layout_passes=False),
    scratch_types=[
        pltpu.VMEM((NPAD,), jnp.float32),
        pltpu.VMEM((NPAD,), jnp.float32),
        pltpu.VMEM((NPAD,), jnp.float32),
        pltpu.VMEM((CPT, B), jnp.int32),
        pltpu.VMEM((CPT, B), jnp.int32),
        pltpu.VMEM((CPT, B), jnp.float32),
        pltpu.VMEM((NS, SLC), jnp.float32),
        pltpu.VMEM_SHARED((NS, NPAD), jnp.float32),
    ],
)
def _edges_a(ai_hbm, aj_hbm, src_hbm, dst_hbm, ex_hbm, den_hbm,
             ai_v, aj_v, den_v, dst_v, src_v, ex_v, red_v, shared_den):
    cid = lax.axis_index("c")
    sid = lax.axis_index("s")
    wid = cid * NS + sid

    pltpu.sync_copy(ai_hbm, ai_v.at[pl.ds(0, N)])
    pltpu.sync_copy(aj_hbm, aj_v.at[pl.ds(0, N)])
    for i in range((NPAD - N) // 16):
        ai_v[pl.ds(N + i * 16, 16)] = jnp.zeros((16,), jnp.float32)
        aj_v[pl.ds(N + i * 16, 16)] = jnp.zeros((16,), jnp.float32)

    def _zero(i, _):
        den_v[pl.ds(i * 16, 16)] = jnp.zeros((16,), jnp.float32)
        return _
    lax.fori_loop(0, NPAD // 16, _zero, None)

    pltpu.sync_copy(dst_hbm.at[pl.ds(wid * CPT, CPT)], dst_v)
    pltpu.sync_copy(src_hbm.at[pl.ds(wid * CPT, CPT)], src_v)

    def _chunk(c, _):
        for g in range(B // 16):
            ds16 = pl.ds(g * 16, 16)
            di = dst_v[c, ds16]
            si = src_v[c, ds16]
            al = plsc.load_gather(ai_v, [di]) + plsc.load_gather(aj_v, [si])
            al = jnp.where(al >= 0.0, al, 0.2 * al)
            exv = jnp.exp(al)
            ex_v[c, ds16] = exv
            plsc.addupdate_scatter(den_v, [di], exv)
        return _
    lax.fori_loop(0, CPT, _chunk, None)
    pltpu.sync_copy(ex_v, ex_hbm.at[pl.ds(wid * CPT, CPT)])

    pltpu.sync_copy(den_v, shared_den.at[sid])
    plsc.subcore_barrier()
    for r in range(NS):
        pltpu.sync_copy(shared_den.at[r, pl.ds(sid * SLC, SLC)], red_v.at[r])
    for i in range(SLC // 16):
        s = red_v[0, pl.ds(i * 16, 16)]
        for r in range(1, NS):
            s = s + red_v[r, pl.ds(i * 16, 16)]
        den_v[pl.ds(i * 16, 16)] = s
    pltpu.sync_copy(den_v.at[pl.ds(0, SLC)],
                    den_hbm.at[pl.ds(cid * NPAD + sid * SLC, SLC)])


@functools.partial(
    pl.kernel,
    out_type=jax.ShapeDtypeStruct((NC * NPADO, D), jnp.float32),
    mesh=_mesh,
    compiler_params=pltpu.CompilerParams(needs_layout_passes=False),
    scratch_types=[
        pltpu.VMEM((S, B), jnp.int32),
        pltpu.VMEM((S, B), jnp.int32),
        pltpu.VMEM((S, B), jnp.float32),
        pltpu.VMEM((S, B), jnp.int32),
        pltpu.VMEM((S, B), jnp.int32),
        pltpu.VMEM((S, B), jnp.float32),
        pltpu.VMEM((B,), jnp.float32),
        pltpu.VMEM((B, D), jnp.float32),
        pltpu.VMEM((B, D), jnp.float32),
        pltpu.VMEM_SHARED((NPADO, D), jnp.float32),
        pltpu.SemaphoreType.DMA,
        pltpu.SemaphoreType.DMA,
        pltpu.SemaphoreType.DMA,
        pltpu.SemaphoreType.DMA,
    ],
)
def _edges_b(ex_hbm, src_hbm, dst_hbm, h_hbm, out_hbm,
             srcA, dstA, exA, srcB, dstB, exB, w_b, rows0_v, rows1_v,
             shared_out, g0, g1, si, so):
    cid = lax.axis_index("c")
    sid = lax.axis_index("s")
    wid = cid * NS + sid

    def _zrows(i, _):
        r = i // (D // 16)
        j = i % (D // 16)
        rows0_v[r, pl.ds(j * 16, 16)] = jnp.zeros((16,), jnp.float32)
        return _
    lax.fori_loop(0, B * (D // 16), _zrows, None)
    for k in range(SLCO // B):
        pltpu.sync_copy(rows0_v, shared_out.at[pl.ds(sid * SLCO + k * B, B)])
    _tail = SLCO - (SLCO // B) * B
    if _tail:
        pltpu.sync_copy(
            rows0_v.at[pl.ds(0, _tail)],
            shared_out.at[pl.ds(sid * SLCO + (SLCO // B) * B, _tail)])
    plsc.subcore_barrier()

    def _issue_idx(s, src_t, dst_t, ex_t):
        row = wid * CPT + s * S
        pltpu.async_copy(src_hbm.at[pl.ds(row, S)], src_t, si)
        pltpu.async_copy(dst_hbm.at[pl.ds(row, S)], dst_t, si)
        pltpu.async_copy(ex_hbm.at[pl.ds(row, S)], ex_t, si)

    def _wait_idx(s, src_t, dst_t, ex_t):
        row = wid * CPT + s * S
        pltpu.make_async_copy(src_hbm.at[pl.ds(row, S)], src_t, si).wait()
        pltpu.make_async_copy(dst_hbm.at[pl.ds(row, S)], dst_t, si).wait()
        pltpu.make_async_copy(ex_hbm.at[pl.ds(row, S)], ex_t, si).wait()

    def _gather(src_t, k, rows_v, gsem):
        pltpu.async_copy(h_hbm.at[src_t.at[k]], rows_v, gsem)

    def _wait_gather(src_t, k, rows_v, gsem):
        pltpu.make_async_copy(h_hbm.at[src_t.at[k]], rows_v, gsem).wait()

    def _scale(k, ex_t, rows_v):
        def _wstage(g, _c):
            ds16 = pl.ds(g * 16, 16)
            w_b[ds16] = ex_t[k, ds16]
            return _c
        lax.fori_loop(0, B // 16, _wstage, None)

        @plsc.parallel_loop(0, B, 1, unroll=16)
        def _edge(e):
            wsplat = plsc.load_gather(
                w_b, [jnp.full((16,), e, jnp.int32)])
            for j in range(D // 16):
                dsj = pl.ds(j * 16, 16)
                rows_v[e, dsj] = rows_v[e, dsj] * wsplat

    def _scale_scatter(k, dst_t, ex_t, rows_v):
        _scale(k, ex_t, rows_v)
        pltpu.sync_copy(rows_v, shared_out.at[dst_t.at[k]], add=True)

    def _scale_scatter_async(k, dst_t, ex_t, rows_v, ssem):
        _scale(k, ex_t, rows_v)
        pltpu.async_copy(rows_v, shared_out.at[dst_t.at[k]], ssem, add=True)

    def _wait_scatter(k, dst_t, rows_v, ssem):
        pltpu.make_async_copy(rows_v, shared_out.at[dst_t.at[k]],
                              ssem).wait()

    def _process_sc(s, X, Y, prefetch, prefetch_idx):
        xs, xd, xe = X

        def _pairs(p, _c):
            a = 2 * p
            _gather(xs, a + 1, rows1_v, g1)
            _wait_gather(xs, a, rows0_v, g0)
            _scale_scatter_async(a, xd, xe, rows0_v, so)
            _wait_gather(xs, a + 1, rows1_v, g1)
            _wait_scatter(a, xd, rows0_v, so)
            _gather(xs, a + 2, rows0_v, g0)
            _scale_scatter(a + 1, xd, xe, rows1_v)
            return _c
        lax.fori_loop(0, S // 2 - 1, _pairs, None)
        _gather(xs, S - 1, rows1_v, g1)
        _wait_gather(xs, S - 2, rows0_v, g0)
        _scale_scatter_async(S - 2, xd, xe, rows0_v, so)
        if prefetch:
            _wait_idx(s + 1, *Y)
        _wait_gather(xs, S - 1, rows1_v, g1)
        _wait_scatter(S - 2, xd, rows0_v, so)
        if prefetch:
            _gather(Y[0], 0, rows0_v, g0)
        _scale_scatter(S - 1, xd, xe, rows1_v)
        if prefetch_idx:
            _issue_idx(s + 2, *X)

    A = (srcA, dstA, exA)
    Bset = (srcB, dstB, exB)
    NSC = CPT // S

    _issue_idx(0, *A)
    _wait_idx(0, *A)
    _issue_idx(1, *Bset)
    _gather(srcA, 0, rows0_v, g0)

    def _scpair(t, _):
        _process_sc(2 * t, A, Bset, True, True)
        _process_sc(2 * t + 1, Bset, A, True, True)
        return _
    lax.fori_loop(0, NSC // 2 - 1, _scpair, None)
    _process_sc(NSC - 2, A, Bset, True, False)
    _process_sc(NSC - 1, Bset, A, False, False)

    plsc.subcore_barrier()
    for k in range(SLCO // B):
        row = sid * SLCO + k * B
        pltpu.sync_copy(shared_out.at[pl.ds(row, B)], rows0_v)
        pltpu.sync_copy(rows0_v, out_hbm.at[pl.ds(cid * NPADO + row, B)])
    if _tail:
        row = sid * SLCO + (SLCO // B) * B
        pltpu.sync_copy(shared_out.at[pl.ds(row, _tail)],
                        rows0_v.at[pl.ds(0, _tail)])
        pltpu.sync_copy(rows0_v.at[pl.ds(0, _tail)],
                        out_hbm.at[pl.ds(cid * NPADO + row, _tail)])


def _bn_body(o_ref, den_ref, g_ref, b_ref, out_ref):
    rec = 1.0 / (den_ref[0, :N] + den_ref[1, :N] + 1e-16)
    o = (o_ref[0, :N, :] + o_ref[1, :N, :]) * rec[:, None]
    mean = jnp.mean(o, axis=0)
    c = o - mean[None, :]
    var = jnp.mean(c * c, axis=0)
    y = c / jnp.sqrt(var + 1e-5)[None, :] * g_ref[0, :][None, :] \
        + b_ref[0, :][None, :]
    out_ref[...] = jnp.maximum(y, 0.0)


def _bn(o2, den2, gamma, beta):
    return pl.pallas_call(
        _bn_body,
        out_shape=jax.ShapeDtypeStruct((N, D), jnp.float32),
    )(o2, den2, gamma, beta)


def kernel(x, edge_index, node_embeddings, W, att_i, att_j, gamma, beta):
    pad_ids = jnp.arange(PADE - E, dtype=jnp.int32)
    src = jnp.concatenate(
        [edge_index[0], pad_ids % N]).reshape(PADC, B)
    dst = jnp.concatenate(
        [edge_index[1], N + pad_ids % (NPADO - N)]).reshape(PADC, B)
    ai2 = att_i.reshape(2, D)
    aj2 = att_j.reshape(2, D)
    h, sai, saj = _prep(x, node_embeddings, W, ai2, aj2)
    sai = sai.reshape(N)
    saj = saj.reshape(N)
    ex, den = _edges_a(sai, saj, src, dst)
    o2 = _edges_b(ex, src, dst, h)
    return _bn(o2.reshape(NC, NPADO, D), den.reshape(NC, NPAD),
               gamma.reshape(1, D), beta.reshape(1, D))


# speedup vs baseline: 1.0135x; 1.0135x over previous
import functools

import jax
import jax.numpy as jnp
from jax import lax
from jax.experimental import pallas as pl
from jax.experimental.pallas import tpu as pltpu
from jax.experimental.pallas import tpu_sc as plsc

N = 10000
E = 320000
D = 128

NC = 2
NS = 16
NW = NC * NS
B = 128
CPT = 80
EPT = CPT * B
PADE = NW * EPT
PADC = PADE // B
S = 8
NPAD = 10240
SLC = NPAD // NS
NPADO = 10112
SLCO = NPADO // NS

_mesh = plsc.VectorSubcoreMesh(core_axis_name="c", subcore_axis_name="s")


_BLK = 1000
_NBLK = N // _BLK


def _prep_body(x_ref, emb_ref, w_ref, ai_ref, aj_ref, h_ref,
               sai_ref, saj_ref):
    h = jnp.dot(x_ref[...], w_ref[...], preferred_element_type=jnp.float32)
    h_ref[...] = h
    e = emb_ref[...]
    sai = jnp.sum(h * ai_ref[0, :][None, :], axis=1) + jnp.sum(
        e * ai_ref[1, :][None, :], axis=1)
    saj = jnp.sum(h * aj_ref[0, :][None, :], axis=1) + jnp.sum(
        e * aj_ref[1, :][None, :], axis=1)
    sai_ref[...] = sai.reshape(1, 1, _BLK)
    saj_ref[...] = saj.reshape(1, 1, _BLK)


def _prep(x, emb, w, ai2, aj2):
    return pl.pallas_call(
        _prep_body,
        grid=(_NBLK,),
        in_specs=[
            pl.BlockSpec((_BLK, D), lambda i: (i, 0)),
            pl.BlockSpec((_BLK, D), lambda i: (i, 0)),
            pl.BlockSpec((D, D), lambda i: (0, 0)),
            pl.BlockSpec((2, D), lambda i: (0, 0)),
            pl.BlockSpec((2, D), lambda i: (0, 0)),
        ],
        out_specs=[
            pl.BlockSpec((_BLK, D), lambda i: (i, 0)),
            pl.BlockSpec((1, 1, _BLK), lambda i: (i, 0, 0)),
            pl.BlockSpec((1, 1, _BLK), lambda i: (i, 0, 0)),
        ],
        out_shape=[
            jax.ShapeDtypeStruct((N, D), jnp.float32),
            jax.ShapeDtypeStruct((_NBLK, 1, _BLK), jnp.float32),
            jax.ShapeDtypeStruct((_NBLK, 1, _BLK), jnp.float32),
        ],
    )(x, emb, w, ai2, aj2)


@functools.partial(
    pl.kernel,
    out_type=[
        jax.ShapeDtypeStruct((PADC, B), jnp.float32),
        jax.ShapeDtypeStruct((NC * NPAD,), jnp.float32),
    ],
    mesh=_mesh,
    compiler_params=pltpu.CompilerParams(needs_layout_passes=False),
    scratch_types=[
        pltpu.VMEM((NPAD,), jnp.float32),
        pltpu.VMEM((NPAD,), jnp.float32),
        pltpu.VMEM((NPAD,), jnp.float32),
        pltpu.VMEM((CPT, B), jnp.int32),
        pltpu.VMEM((CPT, B), jnp.int32),
        pltpu.VMEM((CPT, B), jnp.float32),
        pltpu.VMEM((NS, SLC), jnp.float32),
        pltpu.VMEM_SHARED((NS, NPAD), jnp.float32),
    ],
)
def _edges_a(ai_hbm, aj_hbm, src_hbm, dst_hbm, ex_hbm, den_hbm,
             ai_v, aj_v, den_v, dst_v, src_v, ex_v, red_v, shared_den):
    cid = lax.axis_index("c")
    sid = lax.axis_index("s")
    wid = cid * NS + sid

    pltpu.sync_copy(ai_hbm, ai_v.at[pl.ds(0, N)])
    pltpu.sync_copy(aj_hbm, aj_v.at[pl.ds(0, N)])
    for i in range((NPAD - N) // 16):
        ai_v[pl.ds(N + i * 16, 16)] = jnp.zeros((16,), jnp.float32)
        aj_v[pl.ds(N + i * 16, 16)] = jnp.zeros((16,), jnp.float32)

    def _zero(i, _):
        den_v[pl.ds(i * 16, 16)] = jnp.zeros((16,), jnp.float32)
        return _
    lax.fori_loop(0, NPAD // 16, _zero, None)

    pltpu.sync_copy(dst_hbm.at[pl.ds(wid * CPT, CPT)], dst_v)
    pltpu.sync_copy(src_hbm.at[pl.ds(wid * CPT, CPT)], src_v)

    def _chunk(c, _):
        for g in range(B // 16):
            ds16 = pl.ds(g * 16, 16)
            di = dst_v[c, ds16]
            si = src_v[c, ds16]
            al = plsc.load_gather(ai_v, [di]) + plsc.load_gather(aj_v, [si])
            al = jnp.where(al >= 0.0, al, 0.2 * al)
            exv = jnp.exp(al)
            ex_v[c, ds16] = exv
            plsc.addupdate_scatter(den_v, [di], exv)
        return _
    lax.fori_loop(0, CPT, _chunk, None)
    pltpu.sync_copy(ex_v, ex_hbm.at[pl.ds(wid * CPT, CPT)])

    pltpu.sync_copy(den_v, shared_den.at[sid])
    plsc.subcore_barrier()
    for r in range(NS):
        pltpu.sync_copy(shared_den.at[r, pl.ds(sid * SLC, SLC)], red_v.at[r])
    for i in range(SLC // 16):
        s = red_v[0, pl.ds(i * 16, 16)]
        for r in range(1, NS):
            s = s + red_v[r, pl.ds(i * 16, 16)]
        den_v[pl.ds(i * 16, 16)] = s
    pltpu.sync_copy(den_v.at[pl.ds(0, SLC)],
                    den_hbm.at[pl.ds(cid * NPAD + sid * SLC, SLC)])


@functools.partial(
    pl.kernel,
    out_type=jax.ShapeDtypeStruct((NC * NPADO, D), jnp.float32),
    mesh=_mesh,
    compiler_params=pltpu.CompilerParams(needs_layout_passes=False),
    scratch_types=[
        pltpu.VMEM((S, B), jnp.int32),
        pltpu.VMEM((S, B), jnp.int32),
        pltpu.VMEM((S, B), jnp.float32),
        pltpu.VMEM((S, B), jnp.int32),
        pltpu.VMEM((S, B), jnp.int32),
        pltpu.VMEM((S, B), jnp.float32),
        pltpu.VMEM((B,), jnp.float32),
        pltpu.VMEM((B, D), jnp.float32),
        pltpu.VMEM((B, D), jnp.float32),
        pltpu.VMEM_SHARED((NPADO, D), jnp.float32),
        pltpu.SemaphoreType.DMA,
        pltpu.SemaphoreType.DMA,
        pltpu.SemaphoreType.DMA,
        pltpu.SemaphoreType.DMA,
    ],
)
def _edges_b(ex_hbm, src_hbm, dst_hbm, h_hbm, out_hbm,
             srcA, dstA, exA, srcB, dstB, exB, w_b, rows0_v, rows1_v,
             shared_out, g0, g1, si, so):
    cid = lax.axis_index("c")
    sid = lax.axis_index("s")
    wid = cid * NS + sid

    def _zrows(i, _):
        r = i // (D // 16)
        j = i % (D // 16)
        rows0_v[r, pl.ds(j * 16, 16)] = jnp.zeros((16,), jnp.float32)
        return _
    lax.fori_loop(0, B * (D // 16), _zrows, None)
    for k in range(SLCO // B):
        pltpu.sync_copy(rows0_v, shared_out.at[pl.ds(sid * SLCO + k * B, B)])
    _tail = SLCO - (SLCO // B) * B
    if _tail:
        pltpu.sync_copy(
            rows0_v.at[pl.ds(0, _tail)],
            shared_out.at[pl.ds(sid * SLCO + (SLCO // B) * B, _tail)])
    plsc.subcore_barrier()

    def _issue_idx(s, src_t, dst_t, ex_t):
        row = wid * CPT + s * S
        pltpu.async_copy(src_hbm.at[pl.ds(row, S)], src_t, si)
        pltpu.async_copy(dst_hbm.at[pl.ds(row, S)], dst_t, si)
        pltpu.async_copy(ex_hbm.at[pl.ds(row, S)], ex_t, si)

    def _wait_idx(s, src_t, dst_t, ex_t):
        row = wid * CPT + s * S
        pltpu.make_async_copy(src_hbm.at[pl.ds(row, S)], src_t, si).wait()
        pltpu.make_async_copy(dst_hbm.at[pl.ds(row, S)], dst_t, si).wait()
        pltpu.make_async_copy(ex_hbm.at[pl.ds(row, S)], ex_t, si).wait()

    def _gather(src_t, k, rows_v, gsem):
        pltpu.async_copy(h_hbm.at[src_t.at[k]], rows_v, gsem)

    def _wait_gather(src_t, k, rows_v, gsem):
        pltpu.make_async_copy(h_hbm.at[src_t.at[k]], rows_v, gsem).wait()

    def _scale(k, ex_t, rows_v):
        def _wstage(g, _c):
            ds16 = pl.ds(g * 16, 16)
            w_b[ds16] = ex_t[k, ds16]
            return _c
        lax.fori_loop(0, B // 16, _wstage, None)

        @plsc.parallel_loop(0, B, 1, unroll=8)
        def _edge(e):
            wsplat = plsc.load_gather(
                w_b, [jnp.full((16,), e, jnp.int32)])
            for j in range(D // 16):
                dsj = pl.ds(j * 16, 16)
                rows_v[e, dsj] = rows_v[e, dsj] * wsplat

    def _scale_scatter(k, dst_t, ex_t, rows_v):
        _scale(k, ex_t, rows_v)
        pltpu.sync_copy(rows_v, shared_out.at[dst_t.at[k]], add=True)

    def _scale_scatter_async(k, dst_t, ex_t, rows_v, ssem):
        _scale(k, ex_t, rows_v)
        pltpu.async_copy(rows_v, shared_out.at[dst_t.at[k]], ssem, add=True)

    def _wait_scatter(k, dst_t, rows_v, ssem):
        pltpu.make_async_copy(rows_v, shared_out.at[dst_t.at[k]],
                              ssem).wait()

    def _process_sc(s, X, Y, prefetch, prefetch_idx):
        xs, xd, xe = X

        def _pairs(p, _c):
            a = 2 * p
            _gather(xs, a + 1, rows1_v, g1)
            _wait_gather(xs, a, rows0_v, g0)
            _scale_scatter_async(a, xd, xe, rows0_v, so)
            _wait_gather(xs, a + 1, rows1_v, g1)
            _wait_scatter(a, xd, rows0_v, so)
            _gather(xs, a + 2, rows0_v, g0)
            _scale_scatter(a + 1, xd, xe, rows1_v)
            return _c
        lax.fori_loop(0, S // 2 - 1, _pairs, None)
        _gather(xs, S - 1, rows1_v, g1)
        _wait_gather(xs, S - 2, rows0_v, g0)
        _scale_scatter_async(S - 2, xd, xe, rows0_v, so)
        if prefetch:
            _wait_idx(s + 1, *Y)
        _wait_gather(xs, S - 1, rows1_v, g1)
        _wait_scatter(S - 2, xd, rows0_v, so)
        if prefetch:
            _gather(Y[0], 0, rows0_v, g0)
        _scale_scatter(S - 1, xd, xe, rows1_v)
        if prefetch_idx:
            _issue_idx(s + 2, *X)

    A = (srcA, dstA, exA)
    Bset = (srcB, dstB, exB)
    NSC = CPT // S

    _issue_idx(0, *A)
    _wait_idx(0, *A)
    _issue_idx(1, *Bset)
    _gather(srcA, 0, rows0_v, g0)

    def _scpair(t, _):
        _process_sc(2 * t, A, Bset, True, True)
        _process_sc(2 * t + 1, Bset, A, True, True)
        return _
    lax.fori_loop(0, NSC // 2 - 1, _scpair, None)
    _process_sc(NSC - 2, A, Bset, True, False)
    _process_sc(NSC - 1, Bset, A, False, False)

    plsc.subcore_barrier()
    for k in range(SLCO // B):
        row = sid * SLCO + k * B
        pltpu.sync_copy(shared_out.at[pl.ds(row, B)], rows0_v)
        pltpu.sync_copy(rows0_v, out_hbm.at[pl.ds(cid * NPADO + row, B)])
    if _tail:
        row = sid * SLCO + (SLCO // B) * B
        pltpu.sync_copy(shared_out.at[pl.ds(row, _tail)],
                        rows0_v.at[pl.ds(0, _tail)])
        pltpu.sync_copy(rows0_v.at[pl.ds(0, _tail)],
                        out_hbm.at[pl.ds(cid * NPADO + row, _tail)])


def _bn_body(o_ref, den_ref, g_ref, b_ref, out_ref):
    rec = 1.0 / (den_ref[0, :N] + den_ref[1, :N] + 1e-16)
    o = (o_ref[0, :N, :] + o_ref[1, :N, :]) * rec[:, None]
    mean = jnp.mean(o, axis=0)
    c = o - mean[None, :]
    var = jnp.mean(c * c, axis=0)
    y = c / jnp.sqrt(var + 1e-5)[None, :] * g_ref[0, :][None, :] \
        + b_ref[0, :][None, :]
    out_ref[...] = jnp.maximum(y, 0.0)


def _bn(o2, den2, gamma, beta):
    return pl.pallas_call(
        _bn_body,
        out_shape=jax.ShapeDtypeStruct((N, D), jnp.float32),
    )(o2, den2, gamma, beta)


def kernel(x, edge_index, node_embeddings, W, att_i, att_j, gamma, beta):
    pad_ids = jnp.arange(PADE - E, dtype=jnp.int32)
    src = jnp.concatenate(
        [edge_index[0], pad_ids % N]).reshape(PADC, B)
    dst = jnp.concatenate(
        [edge_index[1], N + pad_ids % (NPADO - N)]).reshape(PADC, B)
    ai2 = att_i.reshape(2, D)
    aj2 = att_j.reshape(2, D)
    h, sai, saj = _prep(x, node_embeddings, W, ai2, aj2)
    sai = sai.reshape(N)
    saj = saj.reshape(N)
    ex, den = _edges_a(sai, saj, src, dst)
    o2 = _edges_b(ex, src, dst, h)
    return _bn(o2.reshape(NC, NPADO, D), den.reshape(NC, NPAD),
               gamma.reshape(1, D), beta.reshape(1, D))
